# Initial kernel scaffold; baseline (speedup 1.0000x reference)
import functools

import jax
import jax.numpy as jnp
from jax import lax
from jax.experimental import pallas as pl
from jax.experimental.pallas import tpu as pltpu
from jax.experimental.pallas import tpu_sc as plsc

NC = 2
NS = 16
L = 16
NW = NC * NS
BLK = 128
CHUNK = 4096


def kernel(batch, W_a, W_b):
    B, H = batch.shape
    total = B * H
    nA, D = W_a.shape
    per_w = total // NW
    n_chunks = per_w // CHUNK
    nvr = CHUNK // L
    maxrow = CHUNK // BLK + 2
    out_rows = total + BLK

    idx_flat = batch.reshape(total).astype(jnp.int32)

    mesh = plsc.VectorSubcoreMesh(core_axis_name="c", subcore_axis_name="s")

    @functools.partial(
        pl.kernel,
        out_type=jax.ShapeDtypeStruct((out_rows, D), jnp.float32),
        mesh=mesh,
        scratch_types=[
            pltpu.VMEM((CHUNK,), jnp.int32),
            pltpu.VMEM((maxrow, BLK), jnp.int32),
            pltpu.VMEM((maxrow, BLK), jnp.int32),
            pltpu.VMEM((maxrow, BLK), jnp.int32),
            pltpu.VMEM((maxrow, BLK), jnp.int32),
            pltpu.VMEM((2, BLK, 64), jnp.float32),
            pltpu.SemaphoreType.DMA,
            pltpu.SemaphoreType.DMA,
        ],
    )
    def sc_kernel(idx_hbm, wa_hbm, wb_hbm, out_hbm,
                  idx_v, idxA, posA, idxB, posB, rows, gsem, ssem):
        wid = lax.axis_index("s") * NC + lax.axis_index("c")
        wbase = wid * per_w
        iota = lax.iota(jnp.int32, L)
        trash = jnp.full((L,), total, jnp.int32)
        zeros = jnp.zeros((L,), jnp.int32)

        def chunk_body(c, _):
            off = wbase + c * CHUNK
            pltpu.sync_copy(idx_hbm.at[pl.ds(off, CHUNK)], idx_v)

            def compact(v, carry):
                offA, offB = carry
                ids = idx_v[pl.ds(v * L, L)]
                maskB = ids >= nA
                maskA = ids < nA
                mA = maskA.astype(jnp.int32)
                inclA = plsc.cumsum(mA)
                exclA = inclA - mA
                cA = jnp.max(inclA)
                posv = off + v * L + iota
                tgtA = offA + exclA
                plsc.store_scatter(idxA, [tgtA >> 7, tgtA & 127], ids,
                                   mask=maskA)
                plsc.store_scatter(posA, [tgtA >> 7, tgtA & 127], posv,
                                   mask=maskA)
                tgtB = offB + (iota - exclA)
                plsc.store_scatter(idxB, [tgtB >> 7, tgtB & 127], ids - nA,
                                   mask=maskB)
                plsc.store_scatter(posB, [tgtB >> 7, tgtB & 127], posv,
                                   mask=maskB)
                return offA + cA, offB + (L - cA)

            cntA, cntB = lax.fori_loop(
                0, nvr, compact, (jnp.int32(0), jnp.int32(0)))

            ka = (cntA + BLK - 1) >> 7
            kb = (cntB + BLK - 1) >> 7
            for t in range(BLK // L):
                tA = cntA + t * L + iota
                mA_ = tA < ka * BLK
                plsc.store_scatter(idxA, [tA >> 7, tA & 127], zeros, mask=mA_)
                plsc.store_scatter(posA, [tA >> 7, tA & 127], trash, mask=mA_)
                tB = cntB + t * L + iota
                mB_ = tB < kb * BLK
                plsc.store_scatter(idxB, [tB >> 7, tB & 127], zeros, mask=mB_)
                plsc.store_scatter(posB, [tB >> 7, tB & 127], trash, mask=mB_)

            def blkA(j, x):
                pltpu.async_copy(wa_hbm.at[idxA.at[j]], rows.at[0], gsem).wait()
                pltpu.async_copy(rows.at[0], out_hbm.at[posA.at[j]], ssem).wait()
                return x

            lax.fori_loop(0, ka, blkA, 0)

            def blkB(j, x):
                pltpu.async_copy(wb_hbm.at[idxB.at[j]], rows.at[1], gsem).wait()
                pltpu.async_copy(rows.at[1], out_hbm.at[posB.at[j]], ssem).wait()
                return x

            lax.fori_loop(0, kb, blkB, 0)
            return 0

        lax.fori_loop(0, n_chunks, chunk_body, 0)

    out = sc_kernel(idx_flat, W_a, W_b)
    return out[:total].reshape(B, H, D)


# speedup vs baseline: 1.8517x; 1.8517x over previous
import functools

import jax
import jax.numpy as jnp
from jax import lax
from jax.experimental import pallas as pl
from jax.experimental.pallas import tpu as pltpu
from jax.experimental.pallas import tpu_sc as plsc

NC = 2
NS = 16
L = 16
NW = NC * NS
BLK = 128
CHUNK = 4096


def kernel(batch, W_a, W_b):
    B, H = batch.shape
    total = B * H
    nA, D = W_a.shape
    per_w = total // NW
    n_chunks = per_w // CHUNK
    nvr = CHUNK // L
    maxrow = CHUNK // BLK + 2
    out_rows = total + BLK

    idx_flat = batch.reshape(total).astype(jnp.int32)

    mesh = plsc.VectorSubcoreMesh(core_axis_name="c", subcore_axis_name="s")

    @functools.partial(
        pl.kernel,
        out_type=jax.ShapeDtypeStruct((out_rows, D), jnp.float32),
        mesh=mesh,
        compiler_params=pltpu.CompilerParams(use_tc_tiling_on_sc=False, needs_layout_passes=False),
        scratch_types=[
            pltpu.VMEM((CHUNK,), jnp.int32),
            pltpu.VMEM((maxrow, BLK), jnp.int32),
            pltpu.VMEM((maxrow, BLK), jnp.int32),
            pltpu.VMEM((maxrow, BLK), jnp.int32),
            pltpu.VMEM((maxrow, BLK), jnp.int32),
            pltpu.VMEM((2, BLK, 64), jnp.float32),
            pltpu.SemaphoreType.DMA,
            pltpu.SemaphoreType.DMA,
        ],
    )
    def sc_kernel(idx_hbm, wa_hbm, wb_hbm, out_hbm,
                  idx_v, idxA, posA, idxB, posB, rows, gsem, ssem):
        wid = lax.axis_index("s") * NC + lax.axis_index("c")
        wbase = wid * per_w
        iota = lax.iota(jnp.int32, L)
        trash = jnp.full((L,), total, jnp.int32)
        zeros = jnp.zeros((L,), jnp.int32)

        def chunk_body(c, _):
            off = wbase + c * CHUNK
            pltpu.sync_copy(idx_hbm.at[pl.ds(off, CHUNK)], idx_v)

            def compact(v, carry):
                offA, offB = carry
                ids = idx_v[pl.ds(v * L, L)]
                maskB = ids >= nA
                maskA = ids < nA
                mA = maskA.astype(jnp.int32)
                inclA = plsc.cumsum(mA)
                exclA = inclA - mA
                cA = jnp.max(inclA)
                posv = off + v * L + iota
                tgtA = offA + exclA
                plsc.store_scatter(idxA, [tgtA >> 7, tgtA & 127], ids,
                                   mask=maskA)
                plsc.store_scatter(posA, [tgtA >> 7, tgtA & 127], posv,
                                   mask=maskA)
                tgtB = offB + (iota - exclA)
                plsc.store_scatter(idxB, [tgtB >> 7, tgtB & 127], ids - nA,
                                   mask=maskB)
                plsc.store_scatter(posB, [tgtB >> 7, tgtB & 127], posv,
                                   mask=maskB)
                return offA + cA, offB + (L - cA)

            cntA, cntB = lax.fori_loop(
                0, nvr, compact, (jnp.int32(0), jnp.int32(0)))

            ka = (cntA + BLK - 1) >> 7
            kb = (cntB + BLK - 1) >> 7
            for t in range(BLK // L):
                tA = cntA + t * L + iota
                mA_ = tA < ka * BLK
                plsc.store_scatter(idxA, [tA >> 7, tA & 127], zeros, mask=mA_)
                plsc.store_scatter(posA, [tA >> 7, tA & 127], trash, mask=mA_)
                tB = cntB + t * L + iota
                mB_ = tB < kb * BLK
                plsc.store_scatter(idxB, [tB >> 7, tB & 127], zeros, mask=mB_)
                plsc.store_scatter(posB, [tB >> 7, tB & 127], trash, mask=mB_)

            def blkA(j, x):
                pltpu.async_copy(wa_hbm.at[idxA.at[j]], rows.at[0], gsem).wait()
                pltpu.async_copy(rows.at[0], out_hbm.at[posA.at[j]], ssem).wait()
                return x

            lax.fori_loop(0, ka, blkA, 0)

            def blkB(j, x):
                pltpu.async_copy(wb_hbm.at[idxB.at[j]], rows.at[1], gsem).wait()
                pltpu.async_copy(rows.at[1], out_hbm.at[posB.at[j]], ssem).wait()
                return x

            lax.fori_loop(0, kb, blkB, 0)
            return 0

        lax.fori_loop(0, n_chunks, chunk_body, 0)

    out = sc_kernel(idx_flat, W_a, W_b)
    return out[:total].reshape(B, H, D)


# speedup vs baseline: 3.3647x; 1.8171x over previous
import functools

import jax
import jax.numpy as jnp
from jax import lax
from jax.experimental import pallas as pl
from jax.experimental.pallas import tpu as pltpu
from jax.experimental.pallas import tpu_sc as plsc

NC = 2
NS = 16
L = 16
NW = NC * NS
BLK = 128
CHUNK = 4096
NBUF = 3


def kernel(batch, W_a, W_b):
    B, H = batch.shape
    total = B * H
    nA, D = W_a.shape
    per_w = total // NW
    n_chunks = per_w // CHUNK
    nvr = CHUNK // L
    maxrow = CHUNK // BLK + 2
    nb_max = CHUNK // BLK + 1

    idx_flat = batch.reshape(total).astype(jnp.int32)

    mesh = plsc.VectorSubcoreMesh(core_axis_name="c", subcore_axis_name="s")

    @functools.partial(
        pl.kernel,
        out_type=jax.ShapeDtypeStruct((total, D), jnp.float32),
        mesh=mesh,
        compiler_params=pltpu.CompilerParams(use_tc_tiling_on_sc=False,
                                             needs_layout_passes=False),
        scratch_types=[
            pltpu.VMEM((CHUNK,), jnp.int32),
            pltpu.VMEM((maxrow, BLK), jnp.int32),
            pltpu.VMEM((maxrow, BLK), jnp.int32),
            pltpu.VMEM((maxrow, BLK), jnp.int32),
            pltpu.VMEM((maxrow, BLK), jnp.int32),
            pltpu.VMEM((NBUF, BLK, 64), jnp.float32),
        ] + [pltpu.SemaphoreType.DMA] * (2 * NBUF),
    )
    def sc_kernel(idx_hbm, wa_hbm, wb_hbm, out_hbm,
                  idx_v, idxA, posA, idxB, posB, rows, *sems):
        gsem = sems[:NBUF]
        ssem = sems[NBUF:]
        wid = lax.axis_index("s") * NC + lax.axis_index("c")
        wbase = wid * per_w
        iota = lax.iota(jnp.int32, L)

        def lane0(vec):
            return jnp.zeros((L,), jnp.int32) + jnp.sum(
                jnp.where(iota == 0, vec, 0))

        def chunk_body(c, _):
            off = wbase + c * CHUNK
            pltpu.sync_copy(idx_hbm.at[pl.ds(off, CHUNK)], idx_v)

            def compact(v, carry):
                offA, offB = carry
                ids = idx_v[pl.ds(v * L, L)]
                maskB = ids >= nA
                maskA = ids < nA
                mA = maskA.astype(jnp.int32)
                inclA = plsc.cumsum(mA)
                exclA = inclA - mA
                cA = jnp.max(inclA)
                posv = off + v * L + iota
                tgtA = offA + exclA
                plsc.store_scatter(idxA, [tgtA >> 7, tgtA & 127], ids,
                                   mask=maskA)
                plsc.store_scatter(posA, [tgtA >> 7, tgtA & 127], posv,
                                   mask=maskA)
                tgtB = offB + (iota - exclA)
                plsc.store_scatter(idxB, [tgtB >> 7, tgtB & 127], ids - nA,
                                   mask=maskB)
                plsc.store_scatter(posB, [tgtB >> 7, tgtB & 127], posv,
                                   mask=maskB)
                return offA + cA, offB + (L - cA)

            cntA, cntB = lax.fori_loop(
                0, nvr, compact, (jnp.int32(0), jnp.int32(0)))

            ka = (cntA + BLK - 1) >> 7
            kb = (cntB + BLK - 1) >> 7
            padidA = lane0(idxA[0, pl.ds(0, L)])
            padposA = lane0(posA[0, pl.ds(0, L)])
            padidB = lane0(idxB[0, pl.ds(0, L)])
            padposB = lane0(posB[0, pl.ds(0, L)])
            for t in range(BLK // L):
                tA = cntA + t * L + iota
                mA_ = tA < ka * BLK
                plsc.store_scatter(idxA, [tA >> 7, tA & 127], padidA,
                                   mask=mA_)
                plsc.store_scatter(posA, [tA >> 7, tA & 127], padposA,
                                   mask=mA_)
                tB = cntB + t * L + iota
                mB_ = tB < kb * BLK
                plsc.store_scatter(idxB, [tB >> 7, tB & 127], padidB,
                                   mask=mB_)
                plsc.store_scatter(posB, [tB >> 7, tB & 127], padposB,
                                   mask=mB_)

            jtot = ka + kb

            def gather_blk(b, slot):
                @pl.when(b < ka)
                def _():
                    pltpu.async_copy(wa_hbm.at[idxA.at[b]], rows.at[slot],
                                     gsem[slot])

                @pl.when(b >= ka)
                def _():
                    pltpu.async_copy(wb_hbm.at[idxB.at[b - ka]],
                                     rows.at[slot], gsem[slot])

            def scatter_blk(b, slot):
                @pl.when(b < ka)
                def _():
                    pltpu.async_copy(rows.at[slot], out_hbm.at[posA.at[b]],
                                     ssem[slot])

                @pl.when(b >= ka)
                def _():
                    pltpu.async_copy(rows.at[slot], out_hbm.at[posB.at[b - ka]],
                                     ssem[slot])

            def wait_gather(slot):
                pltpu.make_async_copy(wa_hbm.at[idxA.at[0]], rows.at[slot],
                                      gsem[slot]).wait()

            def wait_scatter(slot):
                pltpu.make_async_copy(rows.at[slot],
                                      out_hbm.at[posA.at[0]],
                                      ssem[slot]).wait()

            for j in range(nb_max + NBUF - 1):
                g = j
                s = j - (NBUF - 1)
                if g < nb_max:
                    slot = g % NBUF

                    @pl.when(g < jtot)
                    def _(g=g, slot=slot):
                        if g >= NBUF:
                            wait_scatter(slot)
                        gather_blk(g, slot)

                if s >= 0:
                    slot = s % NBUF

                    @pl.when(s < jtot)
                    def _(s=s, slot=slot):
                        wait_gather(slot)
                        scatter_blk(s, slot)

            for slot in range(NBUF):
                wait_scatter(slot)
            return 0

        lax.fori_loop(0, n_chunks, chunk_body, 0)

    out = sc_kernel(idx_flat, W_a, W_b)
    return out.reshape(B, H, D)


# speedup vs baseline: 3.5221x; 1.0468x over previous
import functools

import jax
import jax.numpy as jnp
from jax import lax
from jax.experimental import pallas as pl
from jax.experimental.pallas import tpu as pltpu
from jax.experimental.pallas import tpu_sc as plsc

NC = 2
NS = 16
L = 16
NW = NC * NS
BLK = 128
CHUNK = 5120
NBUF = 4


def kernel(batch, W_a, W_b):
    B, H = batch.shape
    total = B * H
    nA, D = W_a.shape
    per_w = total // NW
    n_chunks = per_w // CHUNK
    nvr = CHUNK // L
    maxrow = CHUNK // BLK + 2
    nb_max = CHUNK // BLK + 1

    idx_flat = batch.reshape(total).astype(jnp.int32)

    mesh = plsc.VectorSubcoreMesh(core_axis_name="c", subcore_axis_name="s")

    @functools.partial(
        pl.kernel,
        out_type=jax.ShapeDtypeStruct((total, D), jnp.float32),
        mesh=mesh,
        compiler_params=pltpu.CompilerParams(use_tc_tiling_on_sc=False,
                                             needs_layout_passes=False),
        scratch_types=[
            pltpu.VMEM((CHUNK,), jnp.int32),
            pltpu.VMEM((maxrow, BLK), jnp.int32),
            pltpu.VMEM((maxrow, BLK), jnp.int32),
            pltpu.VMEM((maxrow, BLK), jnp.int32),
            pltpu.VMEM((maxrow, BLK), jnp.int32),
            pltpu.VMEM((NBUF, BLK, 64), jnp.float32),
        ] + [pltpu.SemaphoreType.DMA] * (2 * NBUF),
    )
    def sc_kernel(idx_hbm, wa_hbm, wb_hbm, out_hbm,
                  idx_v, idxA, posA, idxB, posB, rows, *sems):
        gsem = sems[:NBUF]
        ssem = sems[NBUF:]
        wid = lax.axis_index("s") * NC + lax.axis_index("c")
        wbase = wid * per_w
        iota = lax.iota(jnp.int32, L)

        def lane0(vec):
            return jnp.zeros((L,), jnp.int32) + jnp.sum(
                jnp.where(iota == 0, vec, 0))

        def chunk_body(c, _):
            off = wbase + c * CHUNK
            pltpu.sync_copy(idx_hbm.at[pl.ds(off, CHUNK)], idx_v)

            def compact(v, carry):
                offA, offB = carry
                ids = idx_v[pl.ds(v * L, L)]
                maskB = ids >= nA
                maskA = ids < nA
                mA = maskA.astype(jnp.int32)
                inclA = plsc.cumsum(mA)
                exclA = inclA - mA
                cA = jnp.max(inclA)
                posv = off + v * L + iota
                tgtA = offA + exclA
                plsc.store_scatter(idxA, [tgtA >> 7, tgtA & 127], ids,
                                   mask=maskA)
                plsc.store_scatter(posA, [tgtA >> 7, tgtA & 127], posv,
                                   mask=maskA)
                tgtB = offB + (iota - exclA)
                plsc.store_scatter(idxB, [tgtB >> 7, tgtB & 127], ids - nA,
                                   mask=maskB)
                plsc.store_scatter(posB, [tgtB >> 7, tgtB & 127], posv,
                                   mask=maskB)
                return offA + cA, offB + (L - cA)

            cntA, cntB = lax.fori_loop(
                0, nvr, compact, (jnp.int32(0), jnp.int32(0)))

            ka = (cntA + BLK - 1) >> 7
            kb = (cntB + BLK - 1) >> 7
            padidA = lane0(idxA[0, pl.ds(0, L)])
            padposA = lane0(posA[0, pl.ds(0, L)])
            padidB = lane0(idxB[0, pl.ds(0, L)])
            padposB = lane0(posB[0, pl.ds(0, L)])
            for t in range(BLK // L):
                tA = cntA + t * L + iota
                mA_ = tA < ka * BLK
                plsc.store_scatter(idxA, [tA >> 7, tA & 127], padidA,
                                   mask=mA_)
                plsc.store_scatter(posA, [tA >> 7, tA & 127], padposA,
                                   mask=mA_)
                tB = cntB + t * L + iota
                mB_ = tB < kb * BLK
                plsc.store_scatter(idxB, [tB >> 7, tB & 127], padidB,
                                   mask=mB_)
                plsc.store_scatter(posB, [tB >> 7, tB & 127], padposB,
                                   mask=mB_)

            jtot = ka + kb

            def gather_blk(b, slot):
                @pl.when(b < ka)
                def _():
                    pltpu.async_copy(wa_hbm.at[idxA.at[b]], rows.at[slot],
                                     gsem[slot])

                @pl.when(b >= ka)
                def _():
                    pltpu.async_copy(wb_hbm.at[idxB.at[b - ka]],
                                     rows.at[slot], gsem[slot])

            def scatter_blk(b, slot):
                @pl.when(b < ka)
                def _():
                    pltpu.async_copy(rows.at[slot], out_hbm.at[posA.at[b]],
                                     ssem[slot])

                @pl.when(b >= ka)
                def _():
                    pltpu.async_copy(rows.at[slot], out_hbm.at[posB.at[b - ka]],
                                     ssem[slot])

            def wait_gather(slot):
                pltpu.make_async_copy(wa_hbm.at[idxA.at[0]], rows.at[slot],
                                      gsem[slot]).wait()

            def wait_scatter(slot):
                pltpu.make_async_copy(rows.at[slot],
                                      out_hbm.at[posA.at[0]],
                                      ssem[slot]).wait()

            for j in range(nb_max + NBUF - 1):
                g = j
                s = j - (NBUF - 1)
                if g < nb_max:
                    slot = g % NBUF

                    @pl.when(g < jtot)
                    def _(g=g, slot=slot):
                        if g >= NBUF:
                            wait_scatter(slot)
                        gather_blk(g, slot)

                if s >= 0:
                    slot = s % NBUF

                    @pl.when(s < jtot)
                    def _(s=s, slot=slot):
                        wait_gather(slot)
                        scatter_blk(s, slot)

            for slot in range(NBUF):
                wait_scatter(slot)
            return 0

        lax.fori_loop(0, n_chunks, chunk_body, 0)

    out = sc_kernel(idx_flat, W_a, W_b)
    return out.reshape(B, H, D)
